# baseline (speedup 1.0000x reference)
import jax
import jax.numpy as jnp
from jax import lax
from jax.experimental import pallas as pl
from jax.experimental.pallas import tpu as pltpu
from jax.experimental.pallas import tpu_sc as plsc

_N_NODES = 10000
_N_PAD = 10240
_NROW = 80
_D = 128
_LAMB_INV = 1.0 / 10.0
_L = 16
_NC = 2
_NS = 16
_BLK = 80
_SUPA = 400
_MROW = 128
_PER_TILE = _N_PAD // _NS
_MERGE_W = 10


def _dbuf_loop(nblocks, start_fn, proc_fn):
    start_fn(0, 0)

    def body(i, carry):
        @pl.when(i % 2 == 0)
        def _():
            @pl.when(i + 1 < nblocks)
            def _():
                start_fn(i + 1, 1)
            proc_fn(i, 0)

        @pl.when(i % 2 == 1)
        def _():
            @pl.when(i + 1 < nblocks)
            def _():
                start_fn(i + 1, 0)
            proc_fn(i, 1)

        return carry

    lax.fori_loop(0, nblocks, body, 0)


def _sc_body(nid_hbm, nid3d_hbm, ts_hbm, msg_hbm, part_hbm, tpad_hbm,
             hpad_hbm, na0, na1, tb0, tb1, mb0, mb1, nbB0, nbB1,
             wb, tlb, sem0, sem1, ssem0, ssem1, acc_out, acc_t):
    cid = lax.axis_index("c")
    sid = lax.axis_index("s")
    n_msg = nid_hbm.shape[0]
    ca = n_msg // _NS
    cb = n_msg // (_NC * _NS)
    na_blocks = ca // _SUPA
    nb_blocks = cb // _BLK
    zero16 = jnp.zeros((_L,), jnp.float32)
    nas = (na0, na1)
    tbs = (tb0, tb1)
    mbs = (mb0, mb1)
    sems = (sem0, sem1)
    nbBs = (nbB0, nbB1)
    ssems = (ssem0, ssem1)

    def zrow(r, carry):
        for c in range(_D // _L):
            mb0[r, pl.ds(c * _L, _L)] = zero16
        return carry

    lax.fori_loop(0, _NROW, zrow, 0)

    def a_start(i, buf):
        start = pl.multiple_of(sid * ca + i * _SUPA, 8)
        la = pl.multiple_of(jnp.minimum(start + _SUPA, n_msg - _L), 8)
        pltpu.async_copy(nid_hbm.at[pl.ds(start, _SUPA)],
                         nas[buf].at[pl.ds(0, _SUPA)], sems[buf])
        pltpu.async_copy(nid_hbm.at[pl.ds(la, _L)],
                         nas[buf].at[pl.ds(_SUPA, _L)], sems[buf])
        pltpu.async_copy(ts_hbm.at[pl.ds(start, _SUPA)], tbs[buf], sems[buf])

    def a_proc(i, buf):
        start = pl.multiple_of(sid * ca + i * _SUPA, 8)
        la = pl.multiple_of(jnp.minimum(start + _SUPA, n_msg - _L), 8)
        pltpu.make_async_copy(nid_hbm.at[pl.ds(start, _SUPA)],
                              nas[buf].at[pl.ds(0, _SUPA)], sems[buf]).wait()
        pltpu.make_async_copy(nid_hbm.at[pl.ds(la, _L)],
                              nas[buf].at[pl.ds(_SUPA, _L)], sems[buf]).wait()
        pltpu.make_async_copy(ts_hbm.at[pl.ds(start, _SUPA)], tbs[buf],
                              sems[buf]).wait()

        @pl.when(start + _SUPA >= n_msg)
        def _():
            nas[buf][pl.ds(_SUPA, _L)] = jnp.full((_L,), -1, jnp.int32)

        for g in range(_SUPA // _L):
            cur = nas[buf][pl.ds(g * _L, _L)]
            nxt = nas[buf][pl.ds(g * _L + 1, _L)]
            vals = tbs[buf][pl.ds(g * _L, _L)] + 1.0
            plsc.store_scatter(mb0, [cur >> 7, cur & 127], vals,
                               mask=cur != nxt)

    _dbuf_loop(na_blocks, a_start, a_proc)

    stage0 = pl.multiple_of(sid * _NROW, 8)
    pltpu.sync_copy(mb0.at[pl.ds(0, _NROW), :],
                    acc_out.at[pl.ds(stage0, _NROW), :])
    plsc.subcore_barrier()

    @pl.when(sid < _MERGE_W)
    def _():
        for t in range(_NS):
            pltpu.async_copy(
                acc_out.at[pl.ds(pl.multiple_of(t * _NROW + sid * 8, 8), 8), :],
                mb1.at[pl.ds(t * 8, 8), :], sem0)
        for t in range(_NS):
            pltpu.make_async_copy(
                acc_out.at[pl.ds(pl.multiple_of(t * _NROW + sid * 8, 8), 8), :],
                mb1.at[pl.ds(t * 8, 8), :], sem0).wait()
        def merge_fn(g, carry):
            r = g // (_D // _L)
            c0 = pl.multiple_of((g % (_D // _L)) * _L, _L)
            acc = mb1[r, pl.ds(c0, _L)]
            for t in range(1, _NS):
                acc = acc + mb1[t * 8 + r, pl.ds(c0, _L)]
            mb0[r, pl.ds(c0, _L)] = acc
            mb0[8 + r, pl.ds(c0, _L)] = jnp.maximum(acc - 1.0, 0.0)
            mb0[16 + r, pl.ds(c0, _L)] = jnp.where(acc > 0.0, 1.0, 0.0)
            return carry

        lax.fori_loop(0, 8 * _D // _L, merge_fn, 0)
        mrow = pl.multiple_of(sid * 8, 8)
        pltpu.sync_copy(mb0.at[pl.ds(0, 8), :], acc_t.at[pl.ds(mrow, 8), :])

        @pl.when(cid == 0)
        def _():
            pltpu.sync_copy(mb0.at[pl.ds(8, 8), :],
                            tpad_hbm.at[pl.ds(mrow, 8), :])
            pltpu.sync_copy(mb0.at[pl.ds(16, 8), :],
                            hpad_hbm.at[pl.ds(mrow, 8), :])

    plsc.subcore_barrier()

    pltpu.sync_copy(acc_t, tlb)
    lax.fori_loop(0, _NROW, zrow, 0)
    tile0 = pl.multiple_of(sid * _PER_TILE, 8)
    for k in range(_PER_TILE // _NROW):
        pltpu.sync_copy(mb0.at[pl.ds(0, _NROW), :],
                        acc_out.at[pl.ds(tile0 + k * _NROW, _NROW), :])
    plsc.subcore_barrier()

    def b_start(i, buf):
        if not (isinstance(i, int) and i < 2):
            @pl.when(i >= 2)
            def _():
                pltpu.make_async_copy(mbs[buf].at[pl.ds(0, _BLK), :],
                                      acc_out.at[nbBs[buf].at[0]],
                                      ssems[buf]).wait()
        base = pl.multiple_of(cid * (n_msg // _NC) + sid * cb + i * _BLK, 8)
        pltpu.async_copy(nid3d_hbm.at[base // _BLK], nbBs[buf], sems[buf])
        pltpu.async_copy(ts_hbm.at[pl.ds(base, _BLK)],
                         tbs[buf].at[pl.ds(0, _BLK)], sems[buf])
        pltpu.async_copy(msg_hbm.at[pl.ds(base, _BLK), :],
                         mbs[buf].at[pl.ds(0, _BLK), :], sems[buf])

    def b_proc(i, buf):
        base = pl.multiple_of(cid * (n_msg // _NC) + sid * cb + i * _BLK, 8)
        pltpu.make_async_copy(nid3d_hbm.at[base // _BLK],
                              nbBs[buf], sems[buf]).wait()
        pltpu.make_async_copy(ts_hbm.at[pl.ds(base, _BLK)],
                              tbs[buf].at[pl.ds(0, _BLK)], sems[buf]).wait()
        pltpu.make_async_copy(msg_hbm.at[pl.ds(base, _BLK), :],
                              mbs[buf].at[pl.ds(0, _BLK), :], sems[buf]).wait()
        for g in range(_BLK // _L):
            idxv = nbBs[buf][0, pl.ds(g * _L, _L)]
            tl1 = plsc.load_gather(tlb, [idxv >> 7, idxv & 127])
            w = jnp.exp((tbs[buf][pl.ds(g * _L, _L)] - (tl1 - 1.0)) * _LAMB_INV)
            wb[pl.ds(g * _L, _L)] = w

        @plsc.parallel_loop(0, _BLK, unroll=4)
        def row_fn(r):
            wsp = plsc.load_gather(wb, [lax.broadcast(r, (_L,))])
            for c in range(_D // _L):
                mbs[buf][r, pl.ds(c * _L, _L)] = (
                    mbs[buf][r, pl.ds(c * _L, _L)] * wsp)

        pltpu.async_copy(mbs[buf].at[pl.ds(0, _BLK), :],
                         acc_out.at[nbBs[buf].at[0]], ssems[buf], add=True)

    _dbuf_loop(nb_blocks, b_start, b_proc)
    for buf in range(2):
        pltpu.make_async_copy(mbs[buf].at[pl.ds(0, _BLK), :],
                              acc_out.at[nbBs[buf].at[0]], ssems[buf]).wait()
    plsc.subcore_barrier()

    out0 = pl.multiple_of(sid * _PER_TILE, 8)
    pltpu.sync_copy(acc_out.at[pl.ds(out0, _PER_TILE), :],
                    part_hbm.at[cid, pl.ds(out0, _PER_TILE), :])


@jax.jit
def _sc_call(node_ids, timestamps, messages):
    mesh = plsc.VectorSubcoreMesh(core_axis_name="c", subcore_axis_name="s",
                                  num_cores=_NC, num_subcores=_NS)
    fn = pl.kernel(
        _sc_body,
        out_type=(
            jax.ShapeDtypeStruct((_NC, _N_PAD, _D), jnp.float32),
            jax.ShapeDtypeStruct((_NROW, _D), jnp.float32),
            jax.ShapeDtypeStruct((_NROW, _D), jnp.float32),
        ),
        mesh=mesh,
        compiler_params=pltpu.CompilerParams(needs_layout_passes=False),
        scratch_types=[
            pltpu.VMEM((_SUPA + _L,), jnp.int32),
            pltpu.VMEM((_SUPA + _L,), jnp.int32),
            pltpu.VMEM((_SUPA,), jnp.float32),
            pltpu.VMEM((_SUPA,), jnp.float32),
            pltpu.VMEM((_MROW, _D), jnp.float32),
            pltpu.VMEM((_MROW, _D), jnp.float32),
            pltpu.VMEM((1, _BLK), jnp.int32),
            pltpu.VMEM((1, _BLK), jnp.int32),
            pltpu.VMEM((_BLK,), jnp.float32),
            pltpu.VMEM((_NROW, _D), jnp.float32),
            pltpu.SemaphoreType.DMA,
            pltpu.SemaphoreType.DMA,
            pltpu.SemaphoreType.DMA,
            pltpu.SemaphoreType.DMA,
            pltpu.VMEM_SHARED((_N_PAD, _D), jnp.float32),
            pltpu.VMEM_SHARED((_NROW, _D), jnp.float32),
        ],
    )
    return fn(node_ids, node_ids.reshape(-1, 1, _BLK), timestamps, messages)


def _combine_body(p_ref, o_ref):
    o_ref[...] = p_ref[0] + p_ref[1]


@jax.jit
def _combine(part):
    return pl.pallas_call(
        _combine_body,
        grid=(_N_PAD // _PER_TILE,),
        in_specs=[pl.BlockSpec((_NC, _PER_TILE, _D), lambda i: (0, i, 0))],
        out_specs=pl.BlockSpec((_PER_TILE, _D), lambda i: (i, 0)),
        out_shape=jax.ShapeDtypeStruct((_N_PAD, _D), jnp.float32),
    )(part)


def kernel(node_ids, messages, timestamps):
    part, tpad, hpad = _sc_call(node_ids.astype(jnp.int32),
                                timestamps.astype(jnp.float32),
                                messages)
    out = _combine(part)
    return (hpad.reshape(-1)[:_N_NODES] > 0.5,
            out[:_N_NODES],
            tpad.reshape(-1)[:_N_NODES])
